# baseline (speedup 1.0000x reference)
import functools

import jax
import jax.numpy as jnp
from jax import lax
from jax.experimental import pallas as pl
from jax.experimental.pallas import tpu as pltpu
from jax.experimental.pallas import tpu_sc as plsc

_D = 128
_B = 16384
_NC = 2
_NS = 16
_NW = _NC * _NS
_BPW = _B // _NW

_mesh = plsc.VectorSubcoreMesh(core_axis_name="c", subcore_axis_name="s")


@functools.partial(
    pl.kernel,
    mesh=_mesh,
    out_type=jax.ShapeDtypeStruct((_B, _D), jnp.float32),
    scratch_types=[
        pltpu.VMEM((_BPW,), jnp.int32),
        pltpu.VMEM((_BPW, _D), jnp.float32),
        pltpu.SemaphoreType.DMA,
    ],
)
def _gather_kernel(table_hbm, idx_hbm, out_hbm, idx_v, rows_v, sem):
    wid = lax.axis_index("s") * _NC + lax.axis_index("c")
    base = wid * _BPW
    pltpu.sync_copy(idx_hbm.at[pl.ds(base, _BPW)], idx_v)
    pltpu.async_copy(table_hbm.at[idx_v], rows_v, sem).wait()
    pltpu.sync_copy(rows_v, out_hbm.at[pl.ds(base, _BPW)])


def kernel(is_absent, present_goal_vector, absent_goal_vector):
    table = jnp.stack([present_goal_vector, absent_goal_vector], axis=0)
    idx = is_absent.astype(jnp.int32)
    out = _gather_kernel(table, idx)
    return out[None]


# speedup vs baseline: 12.8128x; 12.8128x over previous
import functools

import jax
import jax.numpy as jnp
from jax import lax
from jax.experimental import pallas as pl
from jax.experimental.pallas import tpu as pltpu
from jax.experimental.pallas import tpu_sc as plsc

_D = 128
_B = 16384
_NC = 2
_NS = 16
_NW = _NC * _NS
_BPW = _B // _NW
_NJ = _D // 16

_mesh = plsc.VectorSubcoreMesh(core_axis_name="c", subcore_axis_name="s")


@functools.partial(
    pl.kernel,
    mesh=_mesh,
    out_type=jax.ShapeDtypeStruct((_B, _D), jnp.float32),
    scratch_types=[
        pltpu.VMEM((2 * _D,), jnp.float32),
        pltpu.VMEM((_BPW,), jnp.int32),
        pltpu.VMEM((_BPW, _D), jnp.float32),
    ],
)
def _select_kernel(table_hbm, idx_hbm, out_hbm, table_v, flags_v, rows_v):
    wid = lax.axis_index("s") * _NC + lax.axis_index("c")
    base = wid * _BPW
    pltpu.sync_copy(table_hbm, table_v)
    pltpu.sync_copy(idx_hbm.at[pl.ds(base, _BPW)], flags_v)
    pres = [table_v[pl.ds(16 * j, 16)] for j in range(_NJ)]
    diff = [table_v[pl.ds(_D + 16 * j, 16)] - pres[j] for j in range(_NJ)]
    lane = [jnp.full((16, 1), l, jnp.int32) for l in range(16)]
    dnums = lax.GatherDimensionNumbers(
        offset_dims=(), collapsed_slice_dims=(0,), start_index_map=(0,))

    def body(g, carry):
        fv = flags_v[pl.ds(g * 16, 16)]
        rbase = g * 16
        for l in range(16):
            bl = lax.gather(fv, lane[l], dnums, (1,),
                            mode=lax.GatherScatterMode.PROMISE_IN_BOUNDS)
            f = bl.astype(jnp.float32)
            for j in range(_NJ):
                rows_v[rbase + l, pl.ds(16 * j, 16)] = pres[j] + f * diff[j]
        return carry

    lax.fori_loop(0, _BPW // 16, body, 0)
    pltpu.sync_copy(rows_v, out_hbm.at[pl.ds(base, _BPW)])


def kernel(is_absent, present_goal_vector, absent_goal_vector):
    table = jnp.concatenate([present_goal_vector, absent_goal_vector])
    idx = is_absent.astype(jnp.int32)
    out = _select_kernel(table, idx)
    return out[None]


# speedup vs baseline: 13.0409x; 1.0178x over previous
import functools

import jax
import jax.numpy as jnp
from jax import lax
from jax.experimental import pallas as pl
from jax.experimental.pallas import tpu as pltpu
from jax.experimental.pallas import tpu_sc as plsc

_D = 128
_B = 16384
_NC = 2
_NS = 16
_NW = _NC * _NS
_BPW = _B // _NW

_mesh = plsc.VectorSubcoreMesh(core_axis_name="c", subcore_axis_name="s")


@functools.partial(
    pl.kernel,
    mesh=_mesh,
    out_type=jax.ShapeDtypeStruct((_B, _D), jnp.float32),
    scratch_types=[
        pltpu.VMEM_SHARED((2, _D), jnp.float32),
        pltpu.VMEM((_BPW,), jnp.int32),
        pltpu.VMEM((_BPW, _D), jnp.float32),
        pltpu.SemaphoreType.DMA,
    ],
)
def _gather_kernel(table_hbm, idx_hbm, out_hbm, table_v, flags_v, rows_v, sem):
    wid = lax.axis_index("s") * _NC + lax.axis_index("c")
    base = wid * _BPW
    pltpu.sync_copy(table_hbm, table_v)
    pltpu.sync_copy(idx_hbm.at[pl.ds(base, _BPW)], flags_v)
    pltpu.async_copy(table_v.at[flags_v], rows_v, sem).wait()
    pltpu.sync_copy(rows_v, out_hbm.at[pl.ds(base, _BPW)])


def kernel(is_absent, present_goal_vector, absent_goal_vector):
    table = jnp.stack([present_goal_vector, absent_goal_vector])
    idx = is_absent.astype(jnp.int32)
    out = _gather_kernel(table, idx)
    return out[None]


# speedup vs baseline: 13.3718x; 1.0254x over previous
import functools

import jax
import jax.numpy as jnp
from jax import lax
from jax.experimental import pallas as pl
from jax.experimental.pallas import tpu as pltpu
from jax.experimental.pallas import tpu_sc as plsc

_D = 128
_B = 16384
_NC = 2
_NS = 16
_NW = _NC * _NS
_BPW = _B // _NW
_NCH = 4
_CH = _BPW // _NCH

_mesh = plsc.VectorSubcoreMesh(core_axis_name="c", subcore_axis_name="s")


@functools.partial(
    pl.kernel,
    mesh=_mesh,
    out_type=jax.ShapeDtypeStruct((_B, _D), jnp.float32),
    scratch_types=[
        pltpu.VMEM_SHARED((2, _D), jnp.float32),
        pltpu.VMEM((_BPW,), jnp.int32),
        pltpu.VMEM((_BPW, _D), jnp.float32),
        pltpu.SemaphoreType.DMA,
        pltpu.SemaphoreType.DMA,
        pltpu.SemaphoreType.DMA,
        pltpu.SemaphoreType.DMA,
        pltpu.SemaphoreType.DMA,
        pltpu.SemaphoreType.DMA,
    ],
)
def _gather_kernel(table_hbm, idx_hbm, out_hbm, table_s, flags_v, rows_v,
                   sem_t, sem_i, g0, g1, g2, g3):
    wid = lax.axis_index("s") * _NC + lax.axis_index("c")
    base = wid * _BPW
    cp_t = pltpu.async_copy(table_hbm, table_s, sem_t)
    cp_i = pltpu.async_copy(idx_hbm.at[pl.ds(base, _BPW)], flags_v, sem_i)
    cp_t.wait()
    cp_i.wait()
    gsem = [g0, g1, g2, g3]
    gathers = []
    for k in range(_NCH):
        gathers.append(pltpu.async_copy(
            table_s.at[flags_v.at[pl.ds(k * _CH, _CH)]],
            rows_v.at[pl.ds(k * _CH, _CH)], gsem[k]))
    outs = []
    for k in range(_NCH):
        gathers[k].wait()
        outs.append(pltpu.async_copy(
            rows_v.at[pl.ds(k * _CH, _CH)],
            out_hbm.at[pl.ds(base + k * _CH, _CH)], sem_t))
    for o in outs:
        o.wait()


def kernel(is_absent, present_goal_vector, absent_goal_vector):
    table = jnp.stack([present_goal_vector, absent_goal_vector])
    idx = is_absent.astype(jnp.int32)
    out = _gather_kernel(table, idx)
    return out[None]
